# baseline (speedup 1.0000x reference)
import functools

import jax
import jax.numpy as jnp
from jax import lax
from jax.experimental import pallas as pl
from jax.experimental.pallas import tpu as pltpu

B_TILE = 256
V_TILE = 4096
SUB_W = 512
N_SUB = V_TILE // SUB_W
TOPK = 10
RUN_W = 128
MERGE_AT = RUN_W - N_SUB * TOPK - TOPK
IMAX = 2**31 - 1


def _body(state_ref, w_ref, b_ref, ie_ref, out_ref,
          s_scr, cv_scr, ci_scr, rv_ref, ri_ref, base_ref,
          *, vocab, v_tiles):
    j = pl.program_id(1)

    @pl.when(j == 0)
    def _init():
        rv_ref[...] = jnp.full_like(rv_ref[...], -jnp.inf)
        ri_ref[...] = jnp.zeros_like(ri_ref[...])
        cv_scr[...] = jnp.full_like(cv_scr[...], -jnp.inf)
        ci_scr[...] = jnp.zeros_like(ci_scr[...])
        base_ref[0] = 0

    x = state_ref[...]
    act = jnp.tanh(jnp.dot(x, w_ref[...], preferred_element_type=jnp.float32)
                   + b_ref[...])
    act = act / jnp.sqrt(jnp.sum(act * act, axis=1, keepdims=True))

    ie = ie_ref[...]
    ien = ie / jnp.sqrt(jnp.sum(ie * ie, axis=0, keepdims=True))

    s = jnp.dot(act, ien, preferred_element_type=jnp.float32)

    last_valid = vocab - (v_tiles - 1) * V_TILE

    @pl.when(j < v_tiles - 1)
    def _store():
        s_scr[...] = s

    @pl.when(j == v_tiles - 1)
    def _store_masked():
        lcol = jax.lax.broadcasted_iota(jnp.int32, s.shape, 1)
        s_scr[...] = jnp.where(lcol < last_valid, s, -jnp.inf)

    li = jax.lax.broadcasted_iota(jnp.int32, (B_TILE, SUB_W), 1)
    lane = jax.lax.broadcasted_iota(jnp.int32, (B_TILE, RUN_W), 1)

    for k in range(N_SUB):
        sv0 = s_scr[:, k * SUB_W:(k + 1) * SUB_W]
        v10 = rv_ref[:, TOPK - 1:TOPK]
        nhit = jnp.sum((sv0 > v10).astype(jnp.int32), axis=1, keepdims=True)
        npass = jnp.minimum(jnp.max(nhit), TOPK)
        base = base_ref[0]
        off = j * V_TILE + k * SUB_W

        def _extract(t, _, k=k, off=off, base=base):
            sv = s_scr[:, k * SUB_W:(k + 1) * SUB_W]
            m = jnp.max(sv, axis=1, keepdims=True)
            pick = jnp.min(jnp.where(sv == m, li, IMAX),
                           axis=1, keepdims=True)
            s_scr[:, k * SUB_W:(k + 1) * SUB_W] = (
                jnp.where(li == pick, -jnp.inf, sv))
            cv_scr[...] = jnp.where(lane == base + t, m, cv_scr[...])
            ci_scr[...] = jnp.where(lane == base + t, pick + off,
                                    ci_scr[...])
            return 0

        lax.fori_loop(0, npass, _extract, 0)
        base_ref[0] = base + npass

    do_merge = (base_ref[0] > MERGE_AT) | (j == v_tiles - 1)

    @pl.when(do_merge)
    def _do_merge():
        def _mstep(t, carry):
            mv, mi = carry
            m = jnp.max(mv, axis=1, keepdims=True)
            pick = jnp.min(jnp.where(mv == m, mi, IMAX),
                           axis=1, keepdims=True)
            rv_ref[...] = jnp.where(lane == t, m, rv_ref[...])
            ri_ref[...] = jnp.where(lane == t, pick, ri_ref[...])
            mv = jnp.where(mi == pick, -jnp.inf, mv)
            return mv, mi

        mv0 = jnp.concatenate([cv_scr[...], rv_ref[...]], axis=1)
        mi0 = jnp.concatenate([ci_scr[...], ri_ref[...]], axis=1)
        rv_ref[...] = jnp.full_like(rv_ref[...], -jnp.inf)
        ri_ref[...] = jnp.zeros_like(ri_ref[...])
        lax.fori_loop(0, TOPK, _mstep, (mv0, mi0))
        cv_scr[...] = jnp.full_like(cv_scr[...], -jnp.inf)
        ci_scr[...] = jnp.zeros_like(ci_scr[...])
        base_ref[0] = 0

    @pl.when(j == v_tiles - 1)
    def _emit():
        out_ref[...] = ri_ref[:, :TOPK]


def kernel(state, W, b, item_embeds):
    batch, state_dim = state.shape
    vocab, dim = item_embeds.shape
    ie_t = item_embeds.T
    b2 = b.reshape(1, dim)

    n_b = batch // B_TILE
    n_v = pl.cdiv(vocab, V_TILE)

    out = pl.pallas_call(
        functools.partial(_body, vocab=vocab, v_tiles=n_v),
        grid=(n_b, n_v),
        in_specs=[
            pl.BlockSpec((B_TILE, state_dim), lambda i, j: (i, 0)),
            pl.BlockSpec((state_dim, dim), lambda i, j: (0, 0)),
            pl.BlockSpec((1, dim), lambda i, j: (0, 0)),
            pl.BlockSpec((dim, V_TILE), lambda i, j: (0, j)),
        ],
        out_specs=pl.BlockSpec((B_TILE, TOPK), lambda i, j: (i, 0)),
        out_shape=jax.ShapeDtypeStruct((batch, TOPK), jnp.int32),
        scratch_shapes=[
            pltpu.VMEM((B_TILE, V_TILE), jnp.float32),
            pltpu.VMEM((B_TILE, RUN_W), jnp.float32),
            pltpu.VMEM((B_TILE, RUN_W), jnp.int32),
            pltpu.VMEM((B_TILE, RUN_W), jnp.float32),
            pltpu.VMEM((B_TILE, RUN_W), jnp.int32),
            pltpu.SMEM((1,), jnp.int32),
        ],
    )(state, W, b2, ie_t)
    return out


# speedup vs baseline: 1.2448x; 1.2448x over previous
import functools

import jax
import jax.numpy as jnp
from jax import lax
from jax.experimental import pallas as pl
from jax.experimental.pallas import tpu as pltpu

B_TILE = 256
V_TILE = 4096
SUB_W = 1024
N_SUB = V_TILE // SUB_W
TOPK = 10
RUN_W = 128
MERGE_AT = RUN_W - N_SUB * TOPK - TOPK
IMAX = 2**31 - 1


def _body(state_ref, w_ref, b_ref, ie_ref, out_ref,
          s_scr, cv_scr, ci_scr, rv_ref, ri_ref, base_ref,
          *, vocab, v_tiles):
    j = pl.program_id(1)

    @pl.when(j == 0)
    def _init():
        rv_ref[...] = jnp.full_like(rv_ref[...], -jnp.inf)
        ri_ref[...] = jnp.zeros_like(ri_ref[...])
        cv_scr[...] = jnp.full_like(cv_scr[...], -jnp.inf)
        ci_scr[...] = jnp.zeros_like(ci_scr[...])
        base_ref[0] = 0

    x = state_ref[...]
    act = jnp.tanh(jnp.dot(x, w_ref[...], preferred_element_type=jnp.float32)
                   + b_ref[...])
    act = act / jnp.sqrt(jnp.sum(act * act, axis=1, keepdims=True))

    ie = ie_ref[...]
    ien = ie / jnp.sqrt(jnp.sum(ie * ie, axis=0, keepdims=True))

    s = jnp.dot(act, ien, preferred_element_type=jnp.float32)

    last_valid = vocab - (v_tiles - 1) * V_TILE

    @pl.when(j < v_tiles - 1)
    def _store():
        s_scr[...] = s

    @pl.when(j == v_tiles - 1)
    def _store_masked():
        lcol = jax.lax.broadcasted_iota(jnp.int32, s.shape, 1)
        s_scr[...] = jnp.where(lcol < last_valid, s, -jnp.inf)

    li = jax.lax.broadcasted_iota(jnp.int32, (B_TILE, SUB_W), 1)
    lane = jax.lax.broadcasted_iota(jnp.int32, (B_TILE, RUN_W), 1)

    v10 = rv_ref[:, TOPK - 1:TOPK]
    for k in range(N_SUB):
        sv0 = s[:, k * SUB_W:(k + 1) * SUB_W]
        nhit = jnp.sum((sv0 > v10).astype(jnp.int32), axis=1, keepdims=True)
        npass = jnp.minimum(jnp.max(nhit), TOPK)
        base = base_ref[0]
        off = j * V_TILE + k * SUB_W

        def _extract(t, _, k=k, off=off, base=base):
            sv = s_scr[:, k * SUB_W:(k + 1) * SUB_W]
            m = jnp.max(sv, axis=1, keepdims=True)
            pick = jnp.min(jnp.where(sv == m, li, IMAX),
                           axis=1, keepdims=True)
            s_scr[:, k * SUB_W:(k + 1) * SUB_W] = (
                jnp.where(li == pick, -jnp.inf, sv))
            cv_scr[...] = jnp.where(lane == base + t, m, cv_scr[...])
            ci_scr[...] = jnp.where(lane == base + t, pick + off,
                                    ci_scr[...])
            return 0

        lax.fori_loop(0, npass, _extract, 0)
        base_ref[0] = base + npass

    do_merge = (base_ref[0] > MERGE_AT) | (j == v_tiles - 1)

    @pl.when(do_merge)
    def _do_merge():
        def _mstep(t, carry):
            mv, mi = carry
            m = jnp.max(mv, axis=1, keepdims=True)
            pick = jnp.min(jnp.where(mv == m, mi, IMAX),
                           axis=1, keepdims=True)
            rv_ref[...] = jnp.where(lane == t, m, rv_ref[...])
            ri_ref[...] = jnp.where(lane == t, pick, ri_ref[...])
            mv = jnp.where(mi == pick, -jnp.inf, mv)
            return mv, mi

        mv0 = jnp.concatenate([cv_scr[...], rv_ref[...]], axis=1)
        mi0 = jnp.concatenate([ci_scr[...], ri_ref[...]], axis=1)
        rv_ref[...] = jnp.full_like(rv_ref[...], -jnp.inf)
        ri_ref[...] = jnp.zeros_like(ri_ref[...])
        lax.fori_loop(0, TOPK, _mstep, (mv0, mi0))
        cv_scr[...] = jnp.full_like(cv_scr[...], -jnp.inf)
        ci_scr[...] = jnp.zeros_like(ci_scr[...])
        base_ref[0] = 0

    @pl.when(j == v_tiles - 1)
    def _emit():
        out_ref[...] = ri_ref[:, :TOPK]


def kernel(state, W, b, item_embeds):
    batch, state_dim = state.shape
    vocab, dim = item_embeds.shape
    ie_t = item_embeds.T
    b2 = b.reshape(1, dim)

    n_b = batch // B_TILE
    n_v = pl.cdiv(vocab, V_TILE)

    out = pl.pallas_call(
        functools.partial(_body, vocab=vocab, v_tiles=n_v),
        grid=(n_b, n_v),
        in_specs=[
            pl.BlockSpec((B_TILE, state_dim), lambda i, j: (i, 0)),
            pl.BlockSpec((state_dim, dim), lambda i, j: (0, 0)),
            pl.BlockSpec((1, dim), lambda i, j: (0, 0)),
            pl.BlockSpec((dim, V_TILE), lambda i, j: (0, j)),
        ],
        out_specs=pl.BlockSpec((B_TILE, TOPK), lambda i, j: (i, 0)),
        out_shape=jax.ShapeDtypeStruct((batch, TOPK), jnp.int32),
        scratch_shapes=[
            pltpu.VMEM((B_TILE, V_TILE), jnp.float32),
            pltpu.VMEM((B_TILE, RUN_W), jnp.float32),
            pltpu.VMEM((B_TILE, RUN_W), jnp.int32),
            pltpu.VMEM((B_TILE, RUN_W), jnp.float32),
            pltpu.VMEM((B_TILE, RUN_W), jnp.int32),
            pltpu.SMEM((1,), jnp.int32),
        ],
    )(state, W, b2, ie_t)
    return out


# speedup vs baseline: 1.5363x; 1.2342x over previous
import functools

import jax
import jax.numpy as jnp
from jax import lax
from jax.experimental import pallas as pl
from jax.experimental.pallas import tpu as pltpu

B_TILE = 512
V_TILE = 4096
SUB_W = 1024
N_SUB = V_TILE // SUB_W
TOPK = 10
RUN_W = 128
MERGE_AT = RUN_W - N_SUB * TOPK - TOPK
IMAX = 2**31 - 1


def _body(state_ref, w_ref, b_ref, ie_ref, out_ref,
          s_scr, cv_scr, ci_scr, rv_ref, ri_ref, base_ref,
          *, vocab, v_tiles):
    j = pl.program_id(1)

    @pl.when(j == 0)
    def _init():
        rv_ref[...] = jnp.full_like(rv_ref[...], -jnp.inf)
        ri_ref[...] = jnp.zeros_like(ri_ref[...])
        cv_scr[...] = jnp.full_like(cv_scr[...], -jnp.inf)
        ci_scr[...] = jnp.zeros_like(ci_scr[...])
        base_ref[0] = 0

    x = state_ref[...]
    act = jnp.tanh(jnp.dot(x, w_ref[...], preferred_element_type=jnp.float32)
                   + b_ref[...])
    act = act / jnp.sqrt(jnp.sum(act * act, axis=1, keepdims=True))

    ie = ie_ref[...]
    ien = ie / jnp.sqrt(jnp.sum(ie * ie, axis=0, keepdims=True))

    s = jnp.dot(act, ien, preferred_element_type=jnp.float32)

    last_valid = vocab - (v_tiles - 1) * V_TILE

    @pl.when(j < v_tiles - 1)
    def _store():
        s_scr[...] = s

    @pl.when(j == v_tiles - 1)
    def _store_masked():
        lcol = jax.lax.broadcasted_iota(jnp.int32, s.shape, 1)
        s_scr[...] = jnp.where(lcol < last_valid, s, -jnp.inf)

    li = jax.lax.broadcasted_iota(jnp.int32, (B_TILE, SUB_W), 1)
    lane = jax.lax.broadcasted_iota(jnp.int32, (B_TILE, RUN_W), 1)

    v10 = rv_ref[:, TOPK - 1:TOPK]
    for k in range(N_SUB):
        sv0 = s[:, k * SUB_W:(k + 1) * SUB_W]
        nhit = jnp.sum((sv0 > v10).astype(jnp.int32), axis=1, keepdims=True)
        npass = jnp.minimum(jnp.max(nhit), TOPK)
        base = base_ref[0]
        off = j * V_TILE + k * SUB_W

        def _extract(t, _, k=k, off=off, base=base):
            sv = s_scr[:, k * SUB_W:(k + 1) * SUB_W]
            m = jnp.max(sv, axis=1, keepdims=True)
            pick = jnp.min(jnp.where(sv == m, li, IMAX),
                           axis=1, keepdims=True)
            s_scr[:, k * SUB_W:(k + 1) * SUB_W] = (
                jnp.where(li == pick, -jnp.inf, sv))
            cv_scr[...] = jnp.where(lane == base + t, m, cv_scr[...])
            ci_scr[...] = jnp.where(lane == base + t, pick + off,
                                    ci_scr[...])
            return 0

        lax.fori_loop(0, npass, _extract, 0)
        base_ref[0] = base + npass

    do_merge = (base_ref[0] > MERGE_AT) | (j == v_tiles - 1)

    @pl.when(do_merge)
    def _do_merge():
        def _mstep(t, carry):
            mv, mi = carry
            m = jnp.max(mv, axis=1, keepdims=True)
            pick = jnp.min(jnp.where(mv == m, mi, IMAX),
                           axis=1, keepdims=True)
            rv_ref[...] = jnp.where(lane == t, m, rv_ref[...])
            ri_ref[...] = jnp.where(lane == t, pick, ri_ref[...])
            mv = jnp.where(mi == pick, -jnp.inf, mv)
            return mv, mi

        mv0 = jnp.concatenate([cv_scr[...], rv_ref[...]], axis=1)
        mi0 = jnp.concatenate([ci_scr[...], ri_ref[...]], axis=1)
        rv_ref[...] = jnp.full_like(rv_ref[...], -jnp.inf)
        ri_ref[...] = jnp.zeros_like(ri_ref[...])
        lax.fori_loop(0, TOPK, _mstep, (mv0, mi0))
        cv_scr[...] = jnp.full_like(cv_scr[...], -jnp.inf)
        ci_scr[...] = jnp.zeros_like(ci_scr[...])
        base_ref[0] = 0

    @pl.when(j == v_tiles - 1)
    def _emit():
        out_ref[...] = ri_ref[:, :TOPK]


def kernel(state, W, b, item_embeds):
    batch, state_dim = state.shape
    vocab, dim = item_embeds.shape
    ie_t = item_embeds.T
    b2 = b.reshape(1, dim)

    n_b = batch // B_TILE
    n_v = pl.cdiv(vocab, V_TILE)

    out = pl.pallas_call(
        functools.partial(_body, vocab=vocab, v_tiles=n_v),
        grid=(n_b, n_v),
        in_specs=[
            pl.BlockSpec((B_TILE, state_dim), lambda i, j: (i, 0)),
            pl.BlockSpec((state_dim, dim), lambda i, j: (0, 0)),
            pl.BlockSpec((1, dim), lambda i, j: (0, 0)),
            pl.BlockSpec((dim, V_TILE), lambda i, j: (0, j)),
        ],
        out_specs=pl.BlockSpec((B_TILE, TOPK), lambda i, j: (i, 0)),
        out_shape=jax.ShapeDtypeStruct((batch, TOPK), jnp.int32),
        scratch_shapes=[
            pltpu.VMEM((B_TILE, V_TILE), jnp.float32),
            pltpu.VMEM((B_TILE, RUN_W), jnp.float32),
            pltpu.VMEM((B_TILE, RUN_W), jnp.int32),
            pltpu.VMEM((B_TILE, RUN_W), jnp.float32),
            pltpu.VMEM((B_TILE, RUN_W), jnp.int32),
            pltpu.SMEM((1,), jnp.int32),
        ],
    )(state, W, b2, ie_t)
    return out


# speedup vs baseline: 1.5543x; 1.0117x over previous
import functools

import jax
import jax.numpy as jnp
from jax import lax
from jax.experimental import pallas as pl
from jax.experimental.pallas import tpu as pltpu

B_TILE = 1024
V_TILE = 4096
SUB_W = 1024
N_SUB = V_TILE // SUB_W
TOPK = 10
RUN_W = 128
MERGE_AT = RUN_W - N_SUB * TOPK - TOPK
IMAX = 2**31 - 1


def _body(state_ref, w_ref, b_ref, ie_ref, out_ref,
          s_scr, cv_scr, ci_scr, rv_ref, ri_ref, base_ref,
          *, vocab, v_tiles):
    j = pl.program_id(1)

    @pl.when(j == 0)
    def _init():
        rv_ref[...] = jnp.full_like(rv_ref[...], -jnp.inf)
        ri_ref[...] = jnp.zeros_like(ri_ref[...])
        cv_scr[...] = jnp.full_like(cv_scr[...], -jnp.inf)
        ci_scr[...] = jnp.zeros_like(ci_scr[...])
        base_ref[0] = 0

    x = state_ref[...]
    act = jnp.tanh(jnp.dot(x, w_ref[...], preferred_element_type=jnp.float32)
                   + b_ref[...])
    act = act / jnp.sqrt(jnp.sum(act * act, axis=1, keepdims=True))

    ie = ie_ref[...]
    ien = ie / jnp.sqrt(jnp.sum(ie * ie, axis=0, keepdims=True))

    s = jnp.dot(act, ien, preferred_element_type=jnp.float32)

    last_valid = vocab - (v_tiles - 1) * V_TILE

    @pl.when(j < v_tiles - 1)
    def _store():
        s_scr[...] = s

    @pl.when(j == v_tiles - 1)
    def _store_masked():
        lcol = jax.lax.broadcasted_iota(jnp.int32, s.shape, 1)
        s_scr[...] = jnp.where(lcol < last_valid, s, -jnp.inf)

    li = jax.lax.broadcasted_iota(jnp.int32, (B_TILE, SUB_W), 1)
    lane = jax.lax.broadcasted_iota(jnp.int32, (B_TILE, RUN_W), 1)

    v10 = rv_ref[:, TOPK - 1:TOPK]
    for k in range(N_SUB):
        sv0 = s[:, k * SUB_W:(k + 1) * SUB_W]
        nhit = jnp.sum((sv0 > v10).astype(jnp.int32), axis=1, keepdims=True)
        npass = jnp.minimum(jnp.max(nhit), TOPK)
        base = base_ref[0]
        off = j * V_TILE + k * SUB_W

        def _extract(t, _, k=k, off=off, base=base):
            sv = s_scr[:, k * SUB_W:(k + 1) * SUB_W]
            m = jnp.max(sv, axis=1, keepdims=True)
            pick = jnp.min(jnp.where(sv == m, li, IMAX),
                           axis=1, keepdims=True)
            s_scr[:, k * SUB_W:(k + 1) * SUB_W] = (
                jnp.where(li == pick, -jnp.inf, sv))
            cv_scr[...] = jnp.where(lane == base + t, m, cv_scr[...])
            ci_scr[...] = jnp.where(lane == base + t, pick + off,
                                    ci_scr[...])
            return 0

        lax.fori_loop(0, npass, _extract, 0)
        base_ref[0] = base + npass

    do_merge = (base_ref[0] > MERGE_AT) | (j == v_tiles - 1)

    @pl.when(do_merge)
    def _do_merge():
        def _mstep(t, carry):
            mv, mi = carry
            m = jnp.max(mv, axis=1, keepdims=True)
            pick = jnp.min(jnp.where(mv == m, mi, IMAX),
                           axis=1, keepdims=True)
            rv_ref[...] = jnp.where(lane == t, m, rv_ref[...])
            ri_ref[...] = jnp.where(lane == t, pick, ri_ref[...])
            mv = jnp.where(mi == pick, -jnp.inf, mv)
            return mv, mi

        mv0 = jnp.concatenate([cv_scr[...], rv_ref[...]], axis=1)
        mi0 = jnp.concatenate([ci_scr[...], ri_ref[...]], axis=1)
        rv_ref[...] = jnp.full_like(rv_ref[...], -jnp.inf)
        ri_ref[...] = jnp.zeros_like(ri_ref[...])
        lax.fori_loop(0, TOPK, _mstep, (mv0, mi0))
        cv_scr[...] = jnp.full_like(cv_scr[...], -jnp.inf)
        ci_scr[...] = jnp.zeros_like(ci_scr[...])
        base_ref[0] = 0

    @pl.when(j == v_tiles - 1)
    def _emit():
        out_ref[...] = ri_ref[:, :TOPK]


def kernel(state, W, b, item_embeds):
    batch, state_dim = state.shape
    vocab, dim = item_embeds.shape
    ie_t = item_embeds.T
    b2 = b.reshape(1, dim)

    n_b = batch // B_TILE
    n_v = pl.cdiv(vocab, V_TILE)

    out = pl.pallas_call(
        functools.partial(_body, vocab=vocab, v_tiles=n_v),
        grid=(n_b, n_v),
        in_specs=[
            pl.BlockSpec((B_TILE, state_dim), lambda i, j: (i, 0)),
            pl.BlockSpec((state_dim, dim), lambda i, j: (0, 0)),
            pl.BlockSpec((1, dim), lambda i, j: (0, 0)),
            pl.BlockSpec((dim, V_TILE), lambda i, j: (0, j)),
        ],
        out_specs=pl.BlockSpec((B_TILE, TOPK), lambda i, j: (i, 0)),
        out_shape=jax.ShapeDtypeStruct((batch, TOPK), jnp.int32),
        scratch_shapes=[
            pltpu.VMEM((B_TILE, V_TILE), jnp.float32),
            pltpu.VMEM((B_TILE, RUN_W), jnp.float32),
            pltpu.VMEM((B_TILE, RUN_W), jnp.int32),
            pltpu.VMEM((B_TILE, RUN_W), jnp.float32),
            pltpu.VMEM((B_TILE, RUN_W), jnp.int32),
            pltpu.SMEM((1,), jnp.int32),
        ],
    )(state, W, b2, ie_t)
    return out
